# baseline (speedup 1.0000x reference)
import functools

import jax
import jax.numpy as jnp
from jax import lax
from jax.experimental import pallas as pl
from jax.experimental.pallas import tpu as pltpu
from jax.experimental.pallas import tpu_sc as plsc

T = 2048
H = 768
E = 8
TOPK = 2
DFF = 384
NG = 4
TG = 2
RSF = 2.5

RB = 128
NB_ROUTED = 40
NB_SHARED = T // RB
NBTOT = NB_ROUTED + NB_SHARED
CAP = NB_ROUTED * RB
CAPTOT = NBTOT * RB

NC = 2
NS = 16
NW = NC * NS
TPW = T // NW
NAUX = 8
AUXW = 16


def _router_body(x_ref, gw_ref, bias_ref, aux_ref, be_ref):
    x = x_ref[...]
    logits = lax.dot_general(
        x, gw_ref[...], (((1,), (1,)), ((), ())),
        preferred_element_type=jnp.float32)
    scores = jax.nn.sigmoid(logits)
    s = scores + bias_ref[...]
    gsum = jnp.concatenate(
        [s[:, 2 * g:2 * g + 1] + s[:, 2 * g + 1:2 * g + 2] for g in range(NG)],
        axis=1)
    grank = jnp.zeros_like(gsum)
    giota = lax.broadcasted_iota(jnp.int32, gsum.shape, 1)
    for j in range(NG):
        cj = gsum[:, j:j + 1]
        grank += ((cj > gsum) | ((cj == gsum) & (j < giota))).astype(jnp.float32)
    gmask = (grank < TG).astype(jnp.float32)
    mask8 = jnp.concatenate(
        [gmask[:, g:g + 1] for g in range(NG) for _ in range(E // NG)], axis=1)
    tmp = jnp.where(mask8 > 0, s, 0.0)
    eiota = lax.broadcasted_iota(jnp.int32, tmp.shape, 1)
    erank = jnp.zeros_like(tmp)
    for j in range(E):
        cj = tmp[:, j:j + 1]
        erank += ((cj > tmp) | ((cj == tmp) & (j < eiota))).astype(jnp.float32)
    h0 = (erank == 0).astype(jnp.float32)
    h1 = (erank == 1).astype(jnp.float32)
    msel = h0 + h1
    wun = msel * scores
    denom = jnp.sum(wun, axis=1, keepdims=True)
    wfull = wun / denom * RSF
    w0 = jnp.sum(h0 * wfull, axis=1, keepdims=True)
    w1 = jnp.sum(h1 * wfull, axis=1, keepdims=True)

    CH = 256
    NCH = T // CH
    ar = lax.broadcasted_iota(jnp.int32, (CH, CH), 0)
    ac = lax.broadcasted_iota(jnp.int32, (CH, CH), 1)
    ltri = (ac < ar).astype(jnp.bfloat16)
    c = msel
    cs_parts = []
    off = jnp.zeros((1, E), jnp.float32)
    for i in range(NCH):
        ci = c[CH * i:CH * (i + 1), :]
        within = lax.dot_general(
            ltri, ci.astype(jnp.bfloat16), (((1,), (0,)), ((), ())),
            preferred_element_type=jnp.float32)
        cs_parts.append(within + off)
        off = off + jnp.sum(ci, axis=0, keepdims=True)
    cs = jnp.concatenate(cs_parts, axis=0)
    cnt = off
    nb = jnp.ceil(cnt / RB)
    erow = lax.broadcasted_iota(jnp.int32, (1, E), 1)
    blk_start = jnp.zeros((1, E), jnp.float32)
    for j in range(E):
        blk_start += jnp.where(erow > j, nb[:, j:j + 1], 0.0)
    startrow = blk_start * RB
    pos_all = startrow + cs
    pos0 = jnp.sum(h0 * pos_all, axis=1, keepdims=True)
    pos1 = jnp.sum(h1 * pos_all, axis=1, keepdims=True)
    aux = jnp.concatenate(
        [pos0, pos1, w0, w1, jnp.zeros((T, NAUX - 4), jnp.float32)], axis=1)
    aux_ref[...] = jnp.transpose(aux)

    biota = lax.broadcasted_iota(jnp.int32, (1, NBTOT), 1)
    bev = jnp.zeros((1, NBTOT), jnp.int32)
    for e in range(1, E):
        bev += (biota >= blk_start[:, e:e + 1].astype(jnp.int32)).astype(jnp.int32)
    be_ref[...] = jnp.where(biota >= NB_ROUTED, E, bev)


def _router_call(x, gate_w, bias2d):
    return pl.pallas_call(
        _router_body,
        grid=(1,),
        in_specs=[
            pl.BlockSpec((T, H), lambda i: (0, 0)),
            pl.BlockSpec((E, H), lambda i: (0, 0)),
            pl.BlockSpec((1, E), lambda i: (0, 0)),
        ],
        out_specs=[
            pl.BlockSpec((NAUX, T), lambda i: (0, 0)),
            pl.BlockSpec((1, NBTOT), lambda i: (0, 0)),
        ],
        out_shape=[
            jax.ShapeDtypeStruct((NAUX, T), jnp.float32),
            jax.ShapeDtypeStruct((1, NBTOT), jnp.int32),
        ],
    )(x, gate_w, bias2d)


def _sc_worker_id():
    return lax.axis_index("s") * NC + lax.axis_index("c")


def _f32_to_i32(src_v, dst_v, n, scale=1.0):
    for c4 in range(n // 16):
        sl = pl.ds(c4 * 16, 16)
        dst_v[sl] = (src_v[sl] * scale).astype(jnp.int32)


def _dispatch_sc(x_hbm, p0_hbm, p1_hbm, w0_hbm, w1_hbm, xbig_hbm, wrep_hbm,
                 pf_v, idx_v, idx16_v, w_v, xbuf, sem):
    wid = _sc_worker_id()
    base = wid * TPW
    pltpu.sync_copy(x_hbm.at[pl.ds(base, TPW), :], xbuf)
    for p_hbm, w_hbm in ((p0_hbm, w0_hbm), (p1_hbm, w1_hbm)):
        pltpu.sync_copy(p_hbm.at[pl.ds(base, TPW)], pf_v)
        _f32_to_i32(pf_v, idx_v, TPW)
        pltpu.async_copy(xbuf, xbig_hbm.at[idx_v], sem).wait()
        _f32_to_i32(pf_v, idx16_v, TPW, scale=float(AUXW))
        pltpu.sync_copy(w_hbm.at[pl.ds(base, TPW)], w_v)
        pltpu.async_copy(w_v, wrep_hbm.at[idx16_v], sem).wait()
    pltpu.sync_copy(xbuf, xbig_hbm.at[pl.ds(CAP + base, TPW), :])


def _dispatch_call(x, p0, p1, w0, w1):
    kfn = pl.kernel(
        _dispatch_sc,
        out_type=[
            jax.ShapeDtypeStruct((CAPTOT, H), jnp.float32),
            jax.ShapeDtypeStruct((CAPTOT * AUXW,), jnp.float32),
        ],
        mesh=plsc.VectorSubcoreMesh(core_axis_name="c", subcore_axis_name="s"),
        scratch_types=[
            pltpu.VMEM((TPW,), jnp.float32),
            pltpu.VMEM((TPW,), jnp.int32),
            pltpu.VMEM((TPW,), jnp.int32),
            pltpu.VMEM((TPW,), jnp.float32),
            pltpu.VMEM((TPW, H), jnp.float32),
            pltpu.SemaphoreType.DMA,
        ],
    )
    return kfn(x, p0, p1, w0, w1)


def _gemm_body(be_ref, xg_ref, wgu_ref, wd_ref, wrep_ref, out_ref):
    b = pl.program_id(0)
    xb = xg_ref[...].astype(jnp.bfloat16)
    gu = lax.dot_general(
        xb, wgu_ref[0], (((1,), (1,)), ((), ())),
        preferred_element_type=jnp.float32)
    g = gu[:, :DFF]
    u = gu[:, DFF:]
    h = (g * jax.nn.sigmoid(g) * u).astype(jnp.bfloat16)
    d = lax.dot_general(
        h, wd_ref[0], (((1,), (1,)), ((), ())),
        preferred_element_type=jnp.float32)
    wcol = wrep_ref[:, 0:1]
    wcol = jnp.where(be_ref[b] == E, 1.0, wcol)
    out_ref[...] = d * wcol


def _gemm_call(be1d, xbig, wgu_all, wd_all, wrep):
    grid_spec = pltpu.PrefetchScalarGridSpec(
        num_scalar_prefetch=1,
        grid=(NBTOT,),
        in_specs=[
            pl.BlockSpec((RB, H), lambda b, be: (b, 0)),
            pl.BlockSpec((1, 2 * DFF, H), lambda b, be: (be[b], 0, 0)),
            pl.BlockSpec((1, H, DFF), lambda b, be: (be[b], 0, 0)),
            pl.BlockSpec((RB, AUXW), lambda b, be: (b, 0)),
        ],
        out_specs=pl.BlockSpec((RB, H), lambda b, be: (b, 0)),
    )
    return pl.pallas_call(
        _gemm_body,
        grid_spec=grid_spec,
        out_shape=jax.ShapeDtypeStruct((CAPTOT, H), jnp.float32),
        compiler_params=pltpu.CompilerParams(
            dimension_semantics=("arbitrary",),
        ),
    )(be1d, xbig, wgu_all, wd_all, wrep)


HC = 32


def _combine_sc(xout_hbm, p0_hbm, p1_hbm, out_hbm, pf_v, idx0_v, idx1_v,
                bufa, bufb, sem):
    wid = _sc_worker_id()
    for half in range(TPW // HC):
        base = wid * TPW + half * HC
        pltpu.sync_copy(p0_hbm.at[pl.ds(base, HC)], pf_v)
        _f32_to_i32(pf_v, idx0_v, HC)
        pltpu.sync_copy(p1_hbm.at[pl.ds(base, HC)], pf_v)
        _f32_to_i32(pf_v, idx1_v, HC)
        pltpu.async_copy(xout_hbm.at[idx0_v], bufa, sem).wait()
        pltpu.async_copy(xout_hbm.at[idx1_v], bufb, sem).wait()

        def _add(r, _):
            for o in range(H // 16):
                sl = pl.ds(o * 16, 16)
                bufa[r, sl] = bufa[r, sl] + bufb[r, sl]
            return 0

        lax.fori_loop(0, HC, _add, 0)
        pltpu.sync_copy(xout_hbm.at[pl.ds(CAP + base, HC), :], bufb)
        lax.fori_loop(0, HC, _add, 0)
        pltpu.sync_copy(bufa, out_hbm.at[pl.ds(base, HC), :])


def _combine_call(xout, p0, p1):
    kfn = pl.kernel(
        _combine_sc,
        out_type=jax.ShapeDtypeStruct((T, H), jnp.float32),
        mesh=plsc.VectorSubcoreMesh(core_axis_name="c", subcore_axis_name="s"),
        scratch_types=[
            pltpu.VMEM((HC,), jnp.float32),
            pltpu.VMEM((HC,), jnp.int32),
            pltpu.VMEM((HC,), jnp.int32),
            pltpu.VMEM((HC, H), jnp.float32),
            pltpu.VMEM((HC, H), jnp.float32),
            pltpu.SemaphoreType.DMA,
        ],
    )
    return kfn(xout, p0, p1)


def kernel(hidden_states, gate_W, e_score_correction_bias, We_gate_up, We_down,
           Ws_gate_up, Ws_down):
    wgu_all = jnp.concatenate(
        [We_gate_up, Ws_gate_up[None]], axis=0).astype(jnp.bfloat16)
    wd_all = jnp.concatenate(
        [We_down, Ws_down[None]], axis=0).astype(jnp.bfloat16)
    bias2d = e_score_correction_bias.reshape(1, E)

    aux, be = _router_call(hidden_states, gate_W, bias2d)
    p0, p1, w0, w1 = aux[0], aux[1], aux[2], aux[3]
    xbig, wrep = _dispatch_call(hidden_states, p0, p1, w0, w1)
    xout = _gemm_call(be.reshape(NBTOT), xbig, wgu_all, wd_all,
                      wrep.reshape(CAPTOT, AUXW))
    return _combine_call(xout, p0, p1)


# speedup vs baseline: 1.1131x; 1.1131x over previous
import functools

import jax
import jax.numpy as jnp
from jax import lax
from jax.experimental import pallas as pl
from jax.experimental.pallas import tpu as pltpu
from jax.experimental.pallas import tpu_sc as plsc

T = 2048
H = 768
E = 8
TOPK = 2
DFF = 384
NG = 4
TG = 2
RSF = 2.5

RB = 128
NBTOT = 40
CAPTOT = NBTOT * RB

NC = 2
NS = 16
NW = NC * NS
TPW = T // NW
NAUX = 8
AUXW = 16


def _router_body(x_ref, gw_ref, bias_ref, aux_ref, be_ref):
    x = x_ref[...]
    logits = lax.dot_general(
        x, gw_ref[...], (((1,), (1,)), ((), ())),
        preferred_element_type=jnp.float32)
    scores = jax.nn.sigmoid(logits)
    s = scores + bias_ref[...]
    gsum = jnp.concatenate(
        [s[:, 2 * g:2 * g + 1] + s[:, 2 * g + 1:2 * g + 2] for g in range(NG)],
        axis=1)
    grank = jnp.zeros_like(gsum)
    giota = lax.broadcasted_iota(jnp.int32, gsum.shape, 1)
    for j in range(NG):
        cj = gsum[:, j:j + 1]
        grank += ((cj > gsum) | ((cj == gsum) & (j < giota))).astype(jnp.float32)
    gmask = (grank < TG).astype(jnp.float32)
    mask8 = jnp.concatenate(
        [gmask[:, g:g + 1] for g in range(NG) for _ in range(E // NG)], axis=1)
    tmp = jnp.where(mask8 > 0, s, 0.0)
    eiota = lax.broadcasted_iota(jnp.int32, tmp.shape, 1)
    erank = jnp.zeros_like(tmp)
    for j in range(E):
        cj = tmp[:, j:j + 1]
        erank += ((cj > tmp) | ((cj == tmp) & (j < eiota))).astype(jnp.float32)
    h0 = (erank == 0).astype(jnp.float32)
    h1 = (erank == 1).astype(jnp.float32)
    msel = h0 + h1
    wun = msel * scores
    denom = jnp.sum(wun, axis=1, keepdims=True)
    wfull = wun / denom * RSF
    w0 = jnp.sum(h0 * wfull, axis=1, keepdims=True)
    w1 = jnp.sum(h1 * wfull, axis=1, keepdims=True)

    CH = 256
    NCH = T // CH
    ar = lax.broadcasted_iota(jnp.int32, (CH, CH), 0)
    ac = lax.broadcasted_iota(jnp.int32, (CH, CH), 1)
    ltri = (ac < ar).astype(jnp.bfloat16)
    c = msel
    cs_parts = []
    off = jnp.zeros((1, E), jnp.float32)
    for i in range(NCH):
        ci = c[CH * i:CH * (i + 1), :]
        within = lax.dot_general(
            ltri, ci.astype(jnp.bfloat16), (((1,), (0,)), ((), ())),
            preferred_element_type=jnp.float32)
        cs_parts.append(within + off)
        off = off + jnp.sum(ci, axis=0, keepdims=True)
    cs = jnp.concatenate(cs_parts, axis=0)
    cnt = off
    nb = jnp.ceil(cnt / RB)
    erow = lax.broadcasted_iota(jnp.int32, (1, E), 1)
    blk_start = jnp.zeros((1, E), jnp.float32)
    for j in range(E):
        blk_start += jnp.where(erow > j, nb[:, j:j + 1], 0.0)
    startrow = blk_start * RB
    pos_all = startrow + cs
    pos0 = jnp.sum(h0 * pos_all, axis=1, keepdims=True)
    pos1 = jnp.sum(h1 * pos_all, axis=1, keepdims=True)
    aux = jnp.concatenate(
        [pos0, pos1, w0, w1, jnp.zeros((T, NAUX - 4), jnp.float32)], axis=1)
    aux_ref[...] = jnp.transpose(aux)

    biota = lax.broadcasted_iota(jnp.int32, (1, NBTOT), 1)
    bev = jnp.zeros((1, NBTOT), jnp.int32)
    for e in range(1, E):
        bev += (biota >= blk_start[:, e:e + 1].astype(jnp.int32)).astype(jnp.int32)
    be_ref[...] = bev


def _router_call(x, gate_w, bias2d):
    return pl.pallas_call(
        _router_body,
        grid=(1,),
        in_specs=[
            pl.BlockSpec((T, H), lambda i: (0, 0)),
            pl.BlockSpec((E, H), lambda i: (0, 0)),
            pl.BlockSpec((1, E), lambda i: (0, 0)),
        ],
        out_specs=[
            pl.BlockSpec((NAUX, T), lambda i: (0, 0)),
            pl.BlockSpec((1, NBTOT), lambda i: (0, 0)),
        ],
        out_shape=[
            jax.ShapeDtypeStruct((NAUX, T), jnp.float32),
            jax.ShapeDtypeStruct((1, NBTOT), jnp.int32),
        ],
    )(x, gate_w, bias2d)


def _sc_worker_id():
    return lax.axis_index("s") * NC + lax.axis_index("c")


def _f32_to_i32(src_v, dst_v, n, scale=1.0):
    for c4 in range(n // 16):
        sl = pl.ds(c4 * 16, 16)
        dst_v[sl] = (src_v[sl] * scale).astype(jnp.int32)


def _dispatch_sc(x_hbm, p0_hbm, p1_hbm, w0_hbm, w1_hbm, xbig_hbm, wrep_hbm,
                 pf_v, idx0_v, idx1_v, idx16_v, w_v, xbuf, sem0, sem1):
    wid = _sc_worker_id()
    base = wid * TPW
    pltpu.sync_copy(x_hbm.at[pl.ds(base, TPW), :], xbuf)
    pltpu.sync_copy(p0_hbm.at[pl.ds(base, TPW)], pf_v)
    _f32_to_i32(pf_v, idx0_v, TPW)
    _f32_to_i32(pf_v, idx16_v, TPW, scale=float(AUXW))
    cp0 = pltpu.async_copy(xbuf, xbig_hbm.at[idx0_v], sem0)
    pltpu.sync_copy(w0_hbm.at[pl.ds(base, TPW)], w_v)
    cw0 = pltpu.async_copy(w_v, wrep_hbm.at[idx16_v], sem1)
    pltpu.sync_copy(p1_hbm.at[pl.ds(base, TPW)], pf_v)
    _f32_to_i32(pf_v, idx1_v, TPW)
    cp1 = pltpu.async_copy(xbuf, xbig_hbm.at[idx1_v], sem0)
    cw0.wait()
    _f32_to_i32(pf_v, idx16_v, TPW, scale=float(AUXW))
    pltpu.sync_copy(w1_hbm.at[pl.ds(base, TPW)], w_v)
    cw1 = pltpu.async_copy(w_v, wrep_hbm.at[idx16_v], sem1)
    cp0.wait()
    cp1.wait()
    cw1.wait()


def _dispatch_call(x, p0, p1, w0, w1):
    kfn = pl.kernel(
        _dispatch_sc,
        out_type=[
            jax.ShapeDtypeStruct((CAPTOT, H), jnp.float32),
            jax.ShapeDtypeStruct((CAPTOT * AUXW,), jnp.float32),
        ],
        mesh=plsc.VectorSubcoreMesh(core_axis_name="c", subcore_axis_name="s"),
        scratch_types=[
            pltpu.VMEM((TPW,), jnp.float32),
            pltpu.VMEM((TPW,), jnp.int32),
            pltpu.VMEM((TPW,), jnp.int32),
            pltpu.VMEM((TPW,), jnp.int32),
            pltpu.VMEM((TPW,), jnp.float32),
            pltpu.VMEM((TPW, H), jnp.float32),
            pltpu.SemaphoreType.DMA,
            pltpu.SemaphoreType.DMA,
        ],
    )
    return kfn(x, p0, p1, w0, w1)


def _gemm_body(be_ref, xg_ref, wgu_ref, wd_ref, wrep_ref, out_ref):
    b = pl.program_id(0)
    xb = xg_ref[...].astype(jnp.bfloat16)
    gu = lax.dot_general(
        xb, wgu_ref[0], (((1,), (1,)), ((), ())),
        preferred_element_type=jnp.float32)
    g = gu[:, :DFF]
    u = gu[:, DFF:]
    h = (g * jax.nn.sigmoid(g) * u).astype(jnp.bfloat16)
    d = lax.dot_general(
        h, wd_ref[0], (((1,), (1,)), ((), ())),
        preferred_element_type=jnp.float32)
    wcol = wrep_ref[:, 0:1]
    out_ref[...] = d * wcol


def _gemm_call(be1d, xbig, wgu_all, wd_all, wrep):
    grid_spec = pltpu.PrefetchScalarGridSpec(
        num_scalar_prefetch=1,
        grid=(NBTOT,),
        in_specs=[
            pl.BlockSpec((RB, H), lambda b, be: (b, 0)),
            pl.BlockSpec((1, 2 * DFF, H), lambda b, be: (be[b], 0, 0)),
            pl.BlockSpec((1, H, DFF), lambda b, be: (be[b], 0, 0)),
            pl.BlockSpec((RB, AUXW), lambda b, be: (b, 0)),
        ],
        out_specs=pl.BlockSpec((RB, H), lambda b, be: (b, 0)),
    )
    return pl.pallas_call(
        _gemm_body,
        grid_spec=grid_spec,
        out_shape=jax.ShapeDtypeStruct((CAPTOT, H), jnp.float32),
        compiler_params=pltpu.CompilerParams(
            dimension_semantics=("arbitrary",),
        ),
    )(be1d, xbig, wgu_all, wd_all, wrep)


def _shared_body(x_ref, wgu_ref, wd_ref, out_ref):
    xb = x_ref[...].astype(jnp.bfloat16)
    gu = lax.dot_general(
        xb, wgu_ref[...], (((1,), (1,)), ((), ())),
        preferred_element_type=jnp.float32)
    g = gu[:, :DFF]
    u = gu[:, DFF:]
    h = (g * jax.nn.sigmoid(g) * u).astype(jnp.bfloat16)
    out_ref[...] = lax.dot_general(
        h, wd_ref[...], (((1,), (1,)), ((), ())),
        preferred_element_type=jnp.float32)


def _shared_call(x, ws_gu, ws_d):
    return pl.pallas_call(
        _shared_body,
        grid=(1,),
        in_specs=[
            pl.BlockSpec((T, H), lambda i: (0, 0)),
            pl.BlockSpec((2 * DFF, H), lambda i: (0, 0)),
            pl.BlockSpec((H, DFF), lambda i: (0, 0)),
        ],
        out_specs=pl.BlockSpec((T, H), lambda i: (0, 0)),
        out_shape=jax.ShapeDtypeStruct((T, H), jnp.float32),
    )(x, ws_gu, ws_d)


HC = 32


def _acc(dst, src):
    def _add(r, _):
        for o in range(H // 16):
            sl = pl.ds(o * 16, 16)
            dst[r, sl] = dst[r, sl] + src[r, sl]
        return 0
    lax.fori_loop(0, HC, _add, 0)


def _combine_sc(xout_hbm, sh_hbm, p0_hbm, p1_hbm, out_hbm,
                pf_v, idx0_v, idx1_v, bufa, bufb, bufc, sem0, sem1, sem2):
    wid = _sc_worker_id()
    for half in range(TPW // HC):
        base = wid * TPW + half * HC
        pltpu.sync_copy(p0_hbm.at[pl.ds(base, HC)], pf_v)
        _f32_to_i32(pf_v, idx0_v, HC)
        pltpu.sync_copy(p1_hbm.at[pl.ds(base, HC)], pf_v)
        _f32_to_i32(pf_v, idx1_v, HC)
        ca = pltpu.async_copy(xout_hbm.at[idx0_v], bufa, sem0)
        cb = pltpu.async_copy(xout_hbm.at[idx1_v], bufb, sem1)
        cc = pltpu.async_copy(sh_hbm.at[pl.ds(base, HC), :], bufc, sem2)
        ca.wait()
        cb.wait()
        _acc(bufa, bufb)
        cc.wait()
        _acc(bufa, bufc)
        pltpu.sync_copy(bufa, out_hbm.at[pl.ds(base, HC), :])


def _combine_call(xout, shared, p0, p1):
    kfn = pl.kernel(
        _combine_sc,
        out_type=jax.ShapeDtypeStruct((T, H), jnp.float32),
        mesh=plsc.VectorSubcoreMesh(core_axis_name="c", subcore_axis_name="s"),
        scratch_types=[
            pltpu.VMEM((HC,), jnp.float32),
            pltpu.VMEM((HC,), jnp.int32),
            pltpu.VMEM((HC,), jnp.int32),
            pltpu.VMEM((HC, H), jnp.float32),
            pltpu.VMEM((HC, H), jnp.float32),
            pltpu.VMEM((HC, H), jnp.float32),
            pltpu.SemaphoreType.DMA,
            pltpu.SemaphoreType.DMA,
            pltpu.SemaphoreType.DMA,
        ],
    )
    return kfn(xout, shared, p0, p1)


def kernel(hidden_states, gate_W, e_score_correction_bias, We_gate_up, We_down,
           Ws_gate_up, Ws_down):
    wgu_all = We_gate_up.astype(jnp.bfloat16)
    wd_all = We_down.astype(jnp.bfloat16)
    bias2d = e_score_correction_bias.reshape(1, E)

    aux, be = _router_call(hidden_states, gate_W, bias2d)
    p0, p1, w0, w1 = aux[0], aux[1], aux[2], aux[3]
    shared = _shared_call(hidden_states, Ws_gate_up.astype(jnp.bfloat16),
                          Ws_down.astype(jnp.bfloat16))
    xbig, wrep = _dispatch_call(hidden_states, p0, p1, w0, w1)
    xout = _gemm_call(be.reshape(NBTOT), xbig, wgu_all, wd_all,
                      wrep.reshape(CAPTOT, AUXW))
    return _combine_call(xout, shared, p0, p1)


# speedup vs baseline: 1.1156x; 1.0022x over previous
import functools

import jax
import jax.numpy as jnp
from jax import lax
from jax.experimental import pallas as pl
from jax.experimental.pallas import tpu as pltpu
from jax.experimental.pallas import tpu_sc as plsc

T = 2048
H = 768
E = 8
TOPK = 2
DFF = 384
NG = 4
TG = 2
RSF = 2.5

RB = 128
NBTOT = 40
CAPTOT = NBTOT * RB

NC = 2
NS = 16
NW = NC * NS
TPW = T // NW
NAUX = 8
AUXW = 16


def _router_body(x_ref, gw_ref, bias_ref, aux_ref, be_ref):
    x = x_ref[...]
    logits = lax.dot_general(
        x, gw_ref[...], (((1,), (1,)), ((), ())),
        preferred_element_type=jnp.float32)
    scores = jax.nn.sigmoid(logits)
    s = scores + bias_ref[...]
    gsum = jnp.concatenate(
        [s[:, 2 * g:2 * g + 1] + s[:, 2 * g + 1:2 * g + 2] for g in range(NG)],
        axis=1)
    grank = jnp.zeros_like(gsum)
    giota = lax.broadcasted_iota(jnp.int32, gsum.shape, 1)
    for j in range(NG):
        cj = gsum[:, j:j + 1]
        grank += ((cj > gsum) | ((cj == gsum) & (j < giota))).astype(jnp.float32)
    gmask = (grank < TG).astype(jnp.float32)
    mask8 = jnp.concatenate(
        [gmask[:, g:g + 1] for g in range(NG) for _ in range(E // NG)], axis=1)
    tmp = jnp.where(mask8 > 0, s, 0.0)
    eiota = lax.broadcasted_iota(jnp.int32, tmp.shape, 1)
    erank = jnp.zeros_like(tmp)
    for j in range(E):
        cj = tmp[:, j:j + 1]
        erank += ((cj > tmp) | ((cj == tmp) & (j < eiota))).astype(jnp.float32)
    h0 = (erank == 0).astype(jnp.float32)
    h1 = (erank == 1).astype(jnp.float32)
    msel = h0 + h1
    wun = msel * scores
    denom = jnp.sum(wun, axis=1, keepdims=True)
    wfull = wun / denom * RSF
    w0 = jnp.sum(h0 * wfull, axis=1, keepdims=True)
    w1 = jnp.sum(h1 * wfull, axis=1, keepdims=True)

    CH = 256
    NCH = T // CH
    ar = lax.broadcasted_iota(jnp.int32, (CH, CH), 0)
    ac = lax.broadcasted_iota(jnp.int32, (CH, CH), 1)
    ltri = (ac < ar).astype(jnp.bfloat16)
    c = msel
    cs_parts = []
    off = jnp.zeros((1, E), jnp.float32)
    for i in range(NCH):
        ci = c[CH * i:CH * (i + 1), :]
        within = lax.dot_general(
            ltri, ci.astype(jnp.bfloat16), (((1,), (0,)), ((), ())),
            preferred_element_type=jnp.float32)
        cs_parts.append(within + off)
        off = off + jnp.sum(ci, axis=0, keepdims=True)
    cs = jnp.concatenate(cs_parts, axis=0)
    cnt = off
    nb = jnp.ceil(cnt / RB)
    erow = lax.broadcasted_iota(jnp.int32, (1, E), 1)
    blk_start = jnp.zeros((1, E), jnp.float32)
    for j in range(E):
        blk_start += jnp.where(erow > j, nb[:, j:j + 1], 0.0)
    startrow = blk_start * RB
    pos_all = startrow + cs
    pos0 = jnp.sum(h0 * pos_all, axis=1, keepdims=True)
    pos1 = jnp.sum(h1 * pos_all, axis=1, keepdims=True)
    aux = jnp.concatenate(
        [pos0, pos1, w0, w1, jnp.zeros((T, NAUX - 4), jnp.float32)], axis=1)
    aux_ref[...] = jnp.transpose(aux)

    biota = lax.broadcasted_iota(jnp.int32, (1, NBTOT), 1)
    bev = jnp.zeros((1, NBTOT), jnp.int32)
    for e in range(1, E):
        bev += (biota >= blk_start[:, e:e + 1].astype(jnp.int32)).astype(jnp.int32)
    be_ref[...] = bev


def _router_call(x, gate_w, bias2d):
    return pl.pallas_call(
        _router_body,
        grid=(1,),
        in_specs=[
            pl.BlockSpec((T, H), lambda i: (0, 0)),
            pl.BlockSpec((E, H), lambda i: (0, 0)),
            pl.BlockSpec((1, E), lambda i: (0, 0)),
        ],
        out_specs=[
            pl.BlockSpec((NAUX, T), lambda i: (0, 0)),
            pl.BlockSpec((1, NBTOT), lambda i: (0, 0)),
        ],
        out_shape=[
            jax.ShapeDtypeStruct((NAUX, T), jnp.float32),
            jax.ShapeDtypeStruct((1, NBTOT), jnp.int32),
        ],
    )(x, gate_w, bias2d)


def _sc_worker_id():
    return lax.axis_index("s") * NC + lax.axis_index("c")


def _f32_to_i32(src_v, dst_v, n, scale=1.0):
    for c4 in range(n // 16):
        sl = pl.ds(c4 * 16, 16)
        dst_v[sl] = (src_v[sl] * scale).astype(jnp.int32)


def _dispatch_sc(x_hbm, p0_hbm, p1_hbm, w0_hbm, w1_hbm, xbig_hbm, wrep_hbm,
                 pf_v, idx0_v, idx1_v, idx16_v, w_v, xbuf, sem0, sem1):
    wid = _sc_worker_id()
    base = wid * TPW
    pltpu.sync_copy(x_hbm.at[pl.ds(base, TPW), :], xbuf)
    pltpu.sync_copy(p0_hbm.at[pl.ds(base, TPW)], pf_v)
    _f32_to_i32(pf_v, idx0_v, TPW)
    _f32_to_i32(pf_v, idx16_v, TPW, scale=float(AUXW))
    cp0 = pltpu.async_copy(xbuf, xbig_hbm.at[idx0_v], sem0)
    pltpu.sync_copy(w0_hbm.at[pl.ds(base, TPW)], w_v)
    cw0 = pltpu.async_copy(w_v, wrep_hbm.at[idx16_v], sem1)
    pltpu.sync_copy(p1_hbm.at[pl.ds(base, TPW)], pf_v)
    _f32_to_i32(pf_v, idx1_v, TPW)
    cp1 = pltpu.async_copy(xbuf, xbig_hbm.at[idx1_v], sem0)
    cw0.wait()
    _f32_to_i32(pf_v, idx16_v, TPW, scale=float(AUXW))
    pltpu.sync_copy(w1_hbm.at[pl.ds(base, TPW)], w_v)
    cw1 = pltpu.async_copy(w_v, wrep_hbm.at[idx16_v], sem1)
    cp0.wait()
    cp1.wait()
    cw1.wait()


def _dispatch_call(x, p0, p1, w0, w1):
    kfn = pl.kernel(
        _dispatch_sc,
        out_type=[
            jax.ShapeDtypeStruct((CAPTOT, H), jnp.float32),
            jax.ShapeDtypeStruct((CAPTOT * AUXW,), jnp.float32),
        ],
        mesh=plsc.VectorSubcoreMesh(core_axis_name="c", subcore_axis_name="s"),
        scratch_types=[
            pltpu.VMEM((TPW,), jnp.float32),
            pltpu.VMEM((TPW,), jnp.int32),
            pltpu.VMEM((TPW,), jnp.int32),
            pltpu.VMEM((TPW,), jnp.int32),
            pltpu.VMEM((TPW,), jnp.float32),
            pltpu.VMEM((TPW, H), jnp.float32),
            pltpu.SemaphoreType.DMA,
            pltpu.SemaphoreType.DMA,
        ],
    )
    return kfn(x, p0, p1, w0, w1)


def _gemm_body(be_ref, xg_ref, wgu_ref, wd_ref, wrep_ref, out_ref):
    b = pl.program_id(0)
    e = be_ref[b]
    xb = xg_ref[...].astype(jnp.bfloat16)
    gu = lax.dot_general(
        xb, wgu_ref[e], (((1,), (1,)), ((), ())),
        preferred_element_type=jnp.float32)
    g = gu[:, :DFF]
    u = gu[:, DFF:]
    h = (g * jax.nn.sigmoid(g) * u).astype(jnp.bfloat16)
    d = lax.dot_general(
        h, wd_ref[e], (((1,), (1,)), ((), ())),
        preferred_element_type=jnp.float32)
    wcol = wrep_ref[:, 0:1]
    out_ref[...] = d * wcol


def _gemm_call(be1d, xbig, wgu_all, wd_all, wrep):
    grid_spec = pltpu.PrefetchScalarGridSpec(
        num_scalar_prefetch=1,
        grid=(NBTOT,),
        in_specs=[
            pl.BlockSpec((RB, H), lambda b, be: (b, 0)),
            pl.BlockSpec((E, 2 * DFF, H), lambda b, be: (0, 0, 0)),
            pl.BlockSpec((E, H, DFF), lambda b, be: (0, 0, 0)),
            pl.BlockSpec((RB, AUXW), lambda b, be: (b, 0)),
        ],
        out_specs=pl.BlockSpec((RB, H), lambda b, be: (b, 0)),
    )
    return pl.pallas_call(
        _gemm_body,
        grid_spec=grid_spec,
        out_shape=jax.ShapeDtypeStruct((CAPTOT, H), jnp.float32),
        compiler_params=pltpu.CompilerParams(
            dimension_semantics=("arbitrary",),
        ),
    )(be1d, xbig, wgu_all, wd_all, wrep)


def _shared_body(x_ref, wgu_ref, wd_ref, out_ref):
    xb = x_ref[...].astype(jnp.bfloat16)
    gu = lax.dot_general(
        xb, wgu_ref[...], (((1,), (1,)), ((), ())),
        preferred_element_type=jnp.float32)
    g = gu[:, :DFF]
    u = gu[:, DFF:]
    h = (g * jax.nn.sigmoid(g) * u).astype(jnp.bfloat16)
    out_ref[...] = lax.dot_general(
        h, wd_ref[...], (((1,), (1,)), ((), ())),
        preferred_element_type=jnp.float32)


def _shared_call(x, ws_gu, ws_d):
    return pl.pallas_call(
        _shared_body,
        grid=(1,),
        in_specs=[
            pl.BlockSpec((T, H), lambda i: (0, 0)),
            pl.BlockSpec((2 * DFF, H), lambda i: (0, 0)),
            pl.BlockSpec((H, DFF), lambda i: (0, 0)),
        ],
        out_specs=pl.BlockSpec((T, H), lambda i: (0, 0)),
        out_shape=jax.ShapeDtypeStruct((T, H), jnp.float32),
    )(x, ws_gu, ws_d)


HC = 32


def _acc(dst, src):
    def _add(r, _):
        for o in range(H // 16):
            sl = pl.ds(o * 16, 16)
            dst[r, sl] = dst[r, sl] + src[r, sl]
        return 0
    lax.fori_loop(0, HC, _add, 0)


def _combine_sc(xout_hbm, sh_hbm, p0_hbm, p1_hbm, out_hbm,
                pf_v, idx0_v, idx1_v, bufa, bufb, bufc, sem0, sem1, sem2):
    wid = _sc_worker_id()
    for half in range(TPW // HC):
        base = wid * TPW + half * HC
        pltpu.sync_copy(p0_hbm.at[pl.ds(base, HC)], pf_v)
        _f32_to_i32(pf_v, idx0_v, HC)
        pltpu.sync_copy(p1_hbm.at[pl.ds(base, HC)], pf_v)
        _f32_to_i32(pf_v, idx1_v, HC)
        ca = pltpu.async_copy(xout_hbm.at[idx0_v], bufa, sem0)
        cb = pltpu.async_copy(xout_hbm.at[idx1_v], bufb, sem1)
        cc = pltpu.async_copy(sh_hbm.at[pl.ds(base, HC), :], bufc, sem2)
        ca.wait()
        cb.wait()
        _acc(bufa, bufb)
        cc.wait()
        _acc(bufa, bufc)
        pltpu.sync_copy(bufa, out_hbm.at[pl.ds(base, HC), :])


def _combine_call(xout, shared, p0, p1):
    kfn = pl.kernel(
        _combine_sc,
        out_type=jax.ShapeDtypeStruct((T, H), jnp.float32),
        mesh=plsc.VectorSubcoreMesh(core_axis_name="c", subcore_axis_name="s"),
        scratch_types=[
            pltpu.VMEM((HC,), jnp.float32),
            pltpu.VMEM((HC,), jnp.int32),
            pltpu.VMEM((HC,), jnp.int32),
            pltpu.VMEM((HC, H), jnp.float32),
            pltpu.VMEM((HC, H), jnp.float32),
            pltpu.VMEM((HC, H), jnp.float32),
            pltpu.SemaphoreType.DMA,
            pltpu.SemaphoreType.DMA,
            pltpu.SemaphoreType.DMA,
        ],
    )
    return kfn(xout, shared, p0, p1)


def kernel(hidden_states, gate_W, e_score_correction_bias, We_gate_up, We_down,
           Ws_gate_up, Ws_down):
    wgu_all = We_gate_up.astype(jnp.bfloat16)
    wd_all = We_down.astype(jnp.bfloat16)
    bias2d = e_score_correction_bias.reshape(1, E)

    aux, be = _router_call(hidden_states, gate_W, bias2d)
    p0, p1, w0, w1 = aux[0], aux[1], aux[2], aux[3]
    shared = _shared_call(hidden_states, Ws_gate_up.astype(jnp.bfloat16),
                          Ws_down.astype(jnp.bfloat16))
    xbig, wrep = _dispatch_call(hidden_states, p0, p1, w0, w1)
    xout = _gemm_call(be.reshape(NBTOT), xbig, wgu_all, wd_all,
                      wrep.reshape(CAPTOT, AUXW))
    return _combine_call(xout, shared, p0, p1)
